# baseline (speedup 1.0000x reference)
import functools

import jax
import jax.numpy as jnp
import numpy as np
from jax import lax
from jax.experimental import pallas as pl
from jax.experimental.pallas import tpu as pltpu
from jax.experimental.pallas import tpu_sc as plsc

B, C, H, W = 32, 384, 24, 24
ROW = H * W
N = B * C * ROW
P = 0.1
NUM_ZERO = int(P * C)

_PERM = np.asarray(
    jax.random.permutation(jax.random.key(42), C - 1)[:NUM_ZERO]
).astype(np.int32)
_PERM_PAD = np.full((48,), _PERM[0], dtype=np.int32)
_PERM_PAD[:NUM_ZERO] = _PERM

NC, NS = 2, 16
NW = NC * NS
ROWS_PER_W = (B * C) // NW
CHUNK_ROWS = 48
CHUNK = CHUNK_ROWS * ROW
NCHUNKS = ROWS_PER_W // CHUNK_ROWS

_mesh = plsc.VectorSubcoreMesh(
    core_axis_name="c", subcore_axis_name="s", num_cores=NC, num_subcores=NS
)


@functools.partial(
    pl.kernel,
    out_type=jax.ShapeDtypeStruct((N,), jnp.float32),
    mesh=_mesh,
    compiler_params=pltpu.CompilerParams(needs_layout_passes=False),
    scratch_types=[
        pltpu.VMEM((CHUNK,), jnp.float32),
        pltpu.VMEM((C,), jnp.float32),
        pltpu.VMEM((48,), jnp.int32),
    ],
)
def _sc_scale(x_hbm, perm_hbm, out_hbm, buf, scale_v, perm_v):
    wid = lax.axis_index("s") * NC + lax.axis_index("c")
    base = wid * ROWS_PER_W * ROW

    ones = jnp.full((16,), 1.0, dtype=jnp.float32)
    for i in range(C // 16):
        scale_v[pl.ds(i * 16, 16)] = ones
    pltpu.sync_copy(perm_hbm, perm_v)
    small = jnp.full((16,), 1e-8, dtype=jnp.float32)
    for j in range(3):
        idx = perm_v[pl.ds(j * 16, 16)]
        plsc.store_scatter(scale_v, [idx], small)

    def chunk_body(i, _):
        off = base + i * CHUNK
        pltpu.sync_copy(x_hbm.at[pl.ds(off, CHUNK)], buf)

        def row_body(r, _):
            c = i * CHUNK_ROWS + r
            svec = plsc.load_gather(scale_v, [jnp.full((16,), c, jnp.int32)])

            def vec_body(v, _):
                o = r * ROW + v * 64
                for u in range(4):
                    buf[pl.ds(o + u * 16, 16)] = (
                        buf[pl.ds(o + u * 16, 16)] * svec
                    )
                return 0

            lax.fori_loop(0, ROW // 64, vec_body, 0, unroll=False)
            return 0

        lax.fori_loop(0, CHUNK_ROWS, row_body, 0, unroll=False)
        pltpu.sync_copy(buf, out_hbm.at[pl.ds(off, CHUNK)])
        return 0

    lax.fori_loop(0, NCHUNKS, chunk_body, 0, unroll=False)


def kernel(x):
    out = _sc_scale(x.reshape(N), jnp.asarray(_PERM_PAD))
    return out.reshape(B, C, H, W)


# speedup vs baseline: 11.9073x; 11.9073x over previous
import functools

import jax
import jax.numpy as jnp
import numpy as np
from jax import lax
from jax.experimental import pallas as pl
from jax.experimental.pallas import tpu as pltpu
from jax.experimental.pallas import tpu_sc as plsc

B, C, H, W = 32, 384, 24, 24
NPOS = B * H * W
P = 0.1
NUM_ZERO = int(P * C)

_PERM = np.asarray(
    jax.random.permutation(jax.random.key(42), C - 1)[:NUM_ZERO]
).astype(np.int32)

_SCALE = np.ones((C,), dtype=np.float32)
_SCALE[_PERM] = 1e-8
_GROUPS = [
    (g, _SCALE[g * 16 : (g + 1) * 16].copy())
    for g in range(C // 16)
    if (_SCALE[g * 16 : (g + 1) * 16] != 1.0).any()
]

NC, NS = 2, 16
NW = NC * NS
ROWS_PER_W = NPOS // NW

CHUNK_ROWS = 72
NCH = ROWS_PER_W // CHUNK_ROWS
NBUF = 4
LOOK = 2

_mesh = plsc.VectorSubcoreMesh(
    core_axis_name="c", subcore_axis_name="s", num_cores=NC, num_subcores=NS
)


@functools.partial(
    pl.kernel,
    out_type=jax.ShapeDtypeStruct((NPOS, C), jnp.float32),
    mesh=_mesh,
    compiler_params=pltpu.CompilerParams(needs_layout_passes=False),
    scratch_types=[
        pltpu.VMEM((CHUNK_ROWS, C), jnp.float32),
        pltpu.VMEM((CHUNK_ROWS, C), jnp.float32),
        pltpu.VMEM((CHUNK_ROWS, C), jnp.float32),
        pltpu.VMEM((CHUNK_ROWS, C), jnp.float32),
        pltpu.VMEM((len(_GROUPS) * 16,), jnp.float32),
        pltpu.SemaphoreType.DMA,
        pltpu.SemaphoreType.DMA,
    ],
)
def _sc_scale(x_hbm, scales_hbm, out_hbm, buf0, buf1, buf2, buf3, scale_v, isem, osem):
    bufs = [buf0, buf1, buf2, buf3]
    wid = lax.axis_index("s") * NC + lax.axis_index("c")
    r0 = wid * ROWS_PER_W

    def start_in(j):
        return pltpu.async_copy(
            x_hbm.at[pl.ds(r0 + j * CHUNK_ROWS, CHUNK_ROWS), :],
            bufs[j % NBUF],
            isem,
        )

    def start_out(j):
        return pltpu.async_copy(
            bufs[j % NBUF],
            out_hbm.at[pl.ds(r0 + j * CHUNK_ROWS, CHUNK_ROWS), :],
            osem,
        )

    pltpu.sync_copy(scales_hbm, scale_v)
    scale_vecs = [
        (g, scale_v[pl.ds(k * 16, 16)]) for k, (g, _) in enumerate(_GROUPS)
    ]
    ins, outs = {}, {}
    for j in range(LOOK):
        ins[j] = start_in(j)
    for i in range(NCH):
        ins[i].wait()
        buf = bufs[i % NBUF]

        def row_body(r, _, buf=buf):
            for g, vec in scale_vecs:
                buf[r, pl.ds(g * 16, 16)] = buf[r, pl.ds(g * 16, 16)] * vec
            return 0

        lax.fori_loop(0, CHUNK_ROWS, row_body, 0, unroll=False)
        outs[i] = start_out(i)
        j = i + LOOK
        if j < NCH:
            if j - NBUF >= 0:
                outs[j - NBUF].wait()
            ins[j] = start_in(j)
    for i in range(NCH - NBUF, NCH):
        outs[i].wait()


_SCALE_TABLE = np.concatenate([vec for _, vec in _GROUPS])


def kernel(x):
    x2 = x.transpose(0, 2, 3, 1).reshape(NPOS, C)
    out = _sc_scale(x2, jnp.asarray(_SCALE_TABLE))
    return out.reshape(B, H, W, C).transpose(0, 3, 1, 2)
